# Initial kernel scaffold; baseline (speedup 1.0000x reference)
import jax
import jax.numpy as jnp
from jax.experimental import pallas as pl
from jax.experimental.pallas import tpu as pltpu

N_NODES = 10000
N_EDGES = 320000
N_GRAPHS = 16
EPS = 1e-5
NEG = -1e30

BE = 4000
NEB = N_EDGES // BE
BN_ = 1000
NNB = N_NODES // BN_


def _pass1_body(g1_ref, g2_ref, w1_ref, b1_ref, s_ref, q_ref, acc_s, acc_q):
    step = pl.program_id(0)

    @pl.when(step == 0)
    def _():
        acc_s[...] = jnp.zeros_like(acc_s)
        acc_q[...] = jnp.zeros_like(acc_q)

    msg = g1_ref[...] - g2_ref[...]
    h = jnp.dot(msg, w1_ref[...], preferred_element_type=jnp.float32)
    h = jnp.maximum(h + b1_ref[...], 0.0)
    hr = h.reshape(BE // 8, 8, 64)
    acc_s[...] += hr.sum(0)
    acc_q[...] += (hr * hr).sum(0)

    @pl.when(step == NEB - 1)
    def _():
        s_ref[...] = acc_s[...]
        q_ref[...] = acc_q[...]


def _pass2_body(g1_ref, g2_ref, w1_ref, b1_ref, w2_ref, b2_ref,
                h2_ref, s_ref, q_ref, acc_s, acc_q):
    step = pl.program_id(0)

    @pl.when(step == 0)
    def _():
        acc_s[...] = jnp.zeros_like(acc_s)
        acc_q[...] = jnp.zeros_like(acc_q)

    msg = g1_ref[...] - g2_ref[...]
    h1 = jnp.dot(msg, w1_ref[...], preferred_element_type=jnp.float32)
    h1 = jnp.maximum(h1 + b1_ref[...], 0.0)
    h2 = jnp.dot(h1, w2_ref[...], preferred_element_type=jnp.float32)
    h2 = jnp.maximum(h2 + b2_ref[...], 0.0)
    h2_ref[...] = h2
    hr = h2.reshape(BE // 8, 8, 128)
    acc_s[...] += hr.sum(0)
    acc_q[...] += (hr * hr).sum(0)

    @pl.when(step == NEB - 1)
    def _():
        s_ref[...] = acc_s[...]
        q_ref[...] = acc_q[...]


def _node_body(agg_ref, p3_ref, w3a_ref, w3b_ref, b3_ref, a2_ref, c2_ref,
               hp_ref, s_ref, q_ref, acc_s, acc_q):
    step = pl.program_id(0)

    @pl.when(step == 0)
    def _():
        acc_s[...] = jnp.zeros_like(acc_s)
        acc_q[...] = jnp.zeros_like(acc_q)

    agg = agg_ref[...]
    aggn = jnp.where(agg > -1e29, a2_ref[...] * agg + c2_ref[...], 0.0)
    pre = jnp.dot(aggn, w3a_ref[...], preferred_element_type=jnp.float32)
    pre = pre + jnp.dot(p3_ref[...], w3b_ref[...],
                        preferred_element_type=jnp.float32)
    hp = jnp.maximum(pre + b3_ref[...], 0.0)
    hp_ref[...] = hp
    hr = hp.reshape(BN_ // 8, 8, 128)
    acc_s[...] += hr.sum(0)
    acc_q[...] += (hr * hr).sum(0)

    @pl.when(step == NNB - 1)
    def _():
        s_ref[...] = acc_s[...]
        q_ref[...] = acc_q[...]


def _head_body(hp_ref, b_ref, a3_ref, c3_ref, w4_ref, b4_ref, g4_ref,
               be4_ref, w5_ref, out_ref):
    hpn = a3_ref[...] * hp_ref[...] + c3_ref[...]
    bid = b_ref[...]
    rows = []
    for g in range(N_GRAPHS):
        rows.append(jnp.where(bid == g, hpn, NEG).max(axis=0, keepdims=True))
    gp = jnp.concatenate(rows, axis=0)
    gp = jnp.where(gp > -1e29, gp, 0.0)
    h4 = jnp.dot(gp, w4_ref[...], preferred_element_type=jnp.float32)
    h4 = jnp.maximum(h4 + b4_ref[...], 0.0)
    m4 = jnp.mean(h4, axis=0, keepdims=True)
    v4 = jnp.mean((h4 - m4) * (h4 - m4), axis=0, keepdims=True)
    h4n = g4_ref[...] * (h4 - m4) * jax.lax.rsqrt(v4 + EPS) + be4_ref[...]
    out_ref[...] = jnp.dot(h4n, w5_ref[...], preferred_element_type=jnp.float32)


def _stats_to_affine(s, q, n, g, be):
    mean = s.sum(0) / n
    var = q.sum(0) / n - mean * mean
    a = g * jax.lax.rsqrt(var + EPS)
    c = be - a * mean
    return a, c


def kernel(x, pos, batch, edge_index, W1, b1, g1, be1, W2, b2, g2, be2,
           W3, b3, g3, be3, W4, b4, g4, be4, W5, b5):
    f32 = jnp.float32
    src = edge_index[0]
    dst = edge_index[1]

    pos3 = jnp.concatenate(
        [jnp.cos(pos[:, 1:2]), jnp.sin(pos[:, 1:2]), pos[:, 0:1]], axis=-1)
    zeros3 = jnp.zeros((N_NODES, 3), f32)
    pad2 = jnp.zeros((N_NODES, 2), f32)
    T1 = jnp.concatenate([x, pos3, pad2], axis=1)
    T2 = jnp.concatenate([zeros3, pos3, pad2], axis=1)
    G1 = T1[src]
    G2 = T2[dst]

    W1p = jnp.zeros((8, 64), f32).at[:6].set(W1)
    b1r = b1[None, :]

    s1, q1 = pl.pallas_call(
        _pass1_body,
        grid=(NEB,),
        in_specs=[
            pl.BlockSpec((BE, 8), lambda i: (i, 0)),
            pl.BlockSpec((BE, 8), lambda i: (i, 0)),
            pl.BlockSpec((8, 64), lambda i: (0, 0)),
            pl.BlockSpec((1, 64), lambda i: (0, 0)),
        ],
        out_specs=[
            pl.BlockSpec((8, 64), lambda i: (0, 0)),
            pl.BlockSpec((8, 64), lambda i: (0, 0)),
        ],
        out_shape=[jax.ShapeDtypeStruct((8, 64), f32)] * 2,
        scratch_shapes=[pltpu.VMEM((8, 64), f32)] * 2,
    )(G1, G2, W1p, b1r)

    a1, c1 = _stats_to_affine(s1, q1, N_EDGES, g1, be1)
    W2e = a1[:, None] * W2
    b2e = (b2 + c1 @ W2)[None, :]

    h2, s2, q2 = pl.pallas_call(
        _pass2_body,
        grid=(NEB,),
        in_specs=[
            pl.BlockSpec((BE, 8), lambda i: (i, 0)),
            pl.BlockSpec((BE, 8), lambda i: (i, 0)),
            pl.BlockSpec((8, 64), lambda i: (0, 0)),
            pl.BlockSpec((1, 64), lambda i: (0, 0)),
            pl.BlockSpec((64, 128), lambda i: (0, 0)),
            pl.BlockSpec((1, 128), lambda i: (0, 0)),
        ],
        out_specs=[
            pl.BlockSpec((BE, 128), lambda i: (i, 0)),
            pl.BlockSpec((8, 128), lambda i: (0, 0)),
            pl.BlockSpec((8, 128), lambda i: (0, 0)),
        ],
        out_shape=[
            jax.ShapeDtypeStruct((N_EDGES, 128), f32),
            jax.ShapeDtypeStruct((8, 128), f32),
            jax.ShapeDtypeStruct((8, 128), f32),
        ],
        scratch_shapes=[pltpu.VMEM((8, 128), f32)] * 2,
    )(G1, G2, W1p, b1r, W2e, b2e)

    a2, c2 = _stats_to_affine(s2, q2, N_EDGES, g2, be2)

    aggraw = jnp.full((N_NODES, 128), NEG, f32)
    aggraw = aggraw.at[dst].max(h2)

    pos3p = jnp.concatenate([pos3, jnp.zeros((N_NODES, 5), f32)], axis=1)
    W3a = W3[:128]
    W3b = jnp.zeros((8, 128), f32).at[:3].set(W3[128:131])

    hp, s3, q3 = pl.pallas_call(
        _node_body,
        grid=(NNB,),
        in_specs=[
            pl.BlockSpec((BN_, 128), lambda i: (i, 0)),
            pl.BlockSpec((BN_, 8), lambda i: (i, 0)),
            pl.BlockSpec((128, 128), lambda i: (0, 0)),
            pl.BlockSpec((8, 128), lambda i: (0, 0)),
            pl.BlockSpec((1, 128), lambda i: (0, 0)),
            pl.BlockSpec((1, 128), lambda i: (0, 0)),
            pl.BlockSpec((1, 128), lambda i: (0, 0)),
        ],
        out_specs=[
            pl.BlockSpec((BN_, 128), lambda i: (i, 0)),
            pl.BlockSpec((8, 128), lambda i: (0, 0)),
            pl.BlockSpec((8, 128), lambda i: (0, 0)),
        ],
        out_shape=[
            jax.ShapeDtypeStruct((N_NODES, 128), f32),
            jax.ShapeDtypeStruct((8, 128), f32),
            jax.ShapeDtypeStruct((8, 128), f32),
        ],
        scratch_shapes=[pltpu.VMEM((8, 128), f32)] * 2,
    )(aggraw, pos3p, W3a, W3b, b3[None, :], a2[None, :], c2[None, :])

    a3, c3 = _stats_to_affine(s3, q3, N_NODES, g3, be3)

    W5p = jnp.zeros((64, 128), f32).at[:, 0].set(W5[:, 0])
    out = pl.pallas_call(
        _head_body,
        grid=(1,),
        in_specs=[
            pl.BlockSpec((N_NODES, 128), lambda i: (0, 0)),
            pl.BlockSpec((N_NODES, 1), lambda i: (0, 0)),
            pl.BlockSpec((1, 128), lambda i: (0, 0)),
            pl.BlockSpec((1, 128), lambda i: (0, 0)),
            pl.BlockSpec((128, 64), lambda i: (0, 0)),
            pl.BlockSpec((1, 64), lambda i: (0, 0)),
            pl.BlockSpec((1, 64), lambda i: (0, 0)),
            pl.BlockSpec((1, 64), lambda i: (0, 0)),
            pl.BlockSpec((64, 128), lambda i: (0, 0)),
        ],
        out_specs=pl.BlockSpec((N_GRAPHS, 128), lambda i: (0, 0)),
        out_shape=jax.ShapeDtypeStruct((N_GRAPHS, 128), f32),
    )(hp, batch[:, None], a3[None, :], c3[None, :], W4, b4[None, :],
      g4[None, :], be4[None, :], W5p)

    return out[:, 0:1] + b5[None, :]


# speedup vs baseline: 1.3207x; 1.3207x over previous
import jax
import jax.numpy as jnp
from jax.experimental import pallas as pl
from jax.experimental.pallas import tpu as pltpu

N_NODES = 10000
N_EDGES = 320000
N_GRAPHS = 16
EPS = 1e-5
NEG = -1e30

BE = 4000
NEB = N_EDGES // BE
BN_ = 1000
NNB = N_NODES // BN_


def _dotb(a, b):
    return jnp.dot(a.astype(jnp.bfloat16), b.astype(jnp.bfloat16),
                   preferred_element_type=jnp.float32)


def _pass1_body(g1_ref, g2_ref, w1_ref, b1_ref, s_ref, q_ref, acc_s, acc_q):
    step = pl.program_id(0)

    @pl.when(step == 0)
    def _():
        acc_s[...] = jnp.zeros_like(acc_s)
        acc_q[...] = jnp.zeros_like(acc_q)

    msg = g1_ref[...] - g2_ref[...]
    h = jnp.maximum(_dotb(msg, w1_ref[...]) + b1_ref[...], 0.0)
    acc_s[...] += jnp.sum(h, axis=0, keepdims=True)
    acc_q[...] += jnp.sum(h * h, axis=0, keepdims=True)

    @pl.when(step == NEB - 1)
    def _():
        s_ref[...] = acc_s[...]
        q_ref[...] = acc_q[...]


def _pass2_body(g1_ref, g2_ref, w1_ref, b1_ref, a1_ref, c1_ref, w2_ref,
                b2_ref, h2_ref, s_ref, q_ref, acc_s, acc_q):
    step = pl.program_id(0)

    @pl.when(step == 0)
    def _():
        acc_s[...] = jnp.zeros_like(acc_s)
        acc_q[...] = jnp.zeros_like(acc_q)

    msg = g1_ref[...] - g2_ref[...]
    h1 = jnp.maximum(_dotb(msg, w1_ref[...]) + b1_ref[...], 0.0)
    h1n = a1_ref[...] * h1 + c1_ref[...]
    h2 = jnp.maximum(_dotb(h1n, w2_ref[...]) + b2_ref[...], 0.0)
    h2_ref[...] = h2
    acc_s[...] += jnp.sum(h2, axis=0, keepdims=True)
    acc_q[...] += jnp.sum(h2 * h2, axis=0, keepdims=True)

    @pl.when(step == NEB - 1)
    def _():
        s_ref[...] = acc_s[...]
        q_ref[...] = acc_q[...]


def _node_body(agg_ref, p3_ref, w3a_ref, w3b_ref, b3_ref, a2_ref, c2_ref,
               hp_ref, s_ref, q_ref, acc_s, acc_q):
    step = pl.program_id(0)

    @pl.when(step == 0)
    def _():
        acc_s[...] = jnp.zeros_like(acc_s)
        acc_q[...] = jnp.zeros_like(acc_q)

    agg = agg_ref[...]
    aggn = jnp.where(agg > -1e29, a2_ref[...] * agg + c2_ref[...], 0.0)
    pre = _dotb(aggn, w3a_ref[...]) + _dotb(p3_ref[...], w3b_ref[...])
    hp = jnp.maximum(pre + b3_ref[...], 0.0)
    hp_ref[...] = hp
    acc_s[...] += jnp.sum(hp, axis=0, keepdims=True)
    acc_q[...] += jnp.sum(hp * hp, axis=0, keepdims=True)

    @pl.when(step == NNB - 1)
    def _():
        s_ref[...] = acc_s[...]
        q_ref[...] = acc_q[...]


def _head_body(hp_ref, b_ref, a3_ref, c3_ref, w4_ref, b4_ref, g4_ref,
               be4_ref, w5_ref, out_ref):
    hpn = a3_ref[...] * hp_ref[...] + c3_ref[...]
    bid = b_ref[...]
    rows = []
    for g in range(N_GRAPHS):
        rows.append(jnp.where(bid == g, hpn, NEG).max(axis=0, keepdims=True))
    gp = jnp.concatenate(rows, axis=0)
    gp = jnp.where(gp > -1e29, gp, 0.0)
    h4 = jnp.maximum(_dotb(gp, w4_ref[...]) + b4_ref[...], 0.0)
    m4 = jnp.mean(h4, axis=0, keepdims=True)
    v4 = jnp.mean((h4 - m4) * (h4 - m4), axis=0, keepdims=True)
    h4n = g4_ref[...] * (h4 - m4) / jnp.sqrt(v4 + EPS) + be4_ref[...]
    out_ref[...] = _dotb(h4n, w5_ref[...])


def _stats_to_affine(s, q, n, g, be):
    mean = s.sum(0) / n
    var = q.sum(0) / n - mean * mean
    a = g / jnp.sqrt(var + EPS)
    c = be - a * mean
    return a, c


def kernel(x, pos, batch, edge_index, W1, b1, g1, be1, W2, b2, g2, be2,
           W3, b3, g3, be3, W4, b4, g4, be4, W5, b5):
    f32 = jnp.float32
    src = edge_index[0]
    dst = edge_index[1]

    pos3 = jnp.concatenate(
        [jnp.cos(pos[:, 1:2]), jnp.sin(pos[:, 1:2]), pos[:, 0:1]], axis=-1)
    zeros3 = jnp.zeros((N_NODES, 3), f32)
    pad2 = jnp.zeros((N_NODES, 2), f32)
    T1 = jnp.concatenate([x, pos3, pad2], axis=1)
    T2 = jnp.concatenate([zeros3, pos3, pad2], axis=1)
    G1 = T1[src]
    G2 = T2[dst]

    W1p = jnp.zeros((8, 64), f32).at[:6].set(W1)
    b1r = b1[None, :]

    s1, q1 = pl.pallas_call(
        _pass1_body,
        grid=(NEB,),
        in_specs=[
            pl.BlockSpec((BE, 8), lambda i: (i, 0)),
            pl.BlockSpec((BE, 8), lambda i: (i, 0)),
            pl.BlockSpec((8, 64), lambda i: (0, 0)),
            pl.BlockSpec((1, 64), lambda i: (0, 0)),
        ],
        out_specs=[
            pl.BlockSpec((1, 64), lambda i: (0, 0)),
            pl.BlockSpec((1, 64), lambda i: (0, 0)),
        ],
        out_shape=[jax.ShapeDtypeStruct((1, 64), f32)] * 2,
        scratch_shapes=[pltpu.VMEM((1, 64), f32)] * 2,
    )(G1, G2, W1p, b1r)

    a1, c1 = _stats_to_affine(s1, q1, N_EDGES, g1, be1)

    h2, s2, q2 = pl.pallas_call(
        _pass2_body,
        grid=(NEB,),
        in_specs=[
            pl.BlockSpec((BE, 8), lambda i: (i, 0)),
            pl.BlockSpec((BE, 8), lambda i: (i, 0)),
            pl.BlockSpec((8, 64), lambda i: (0, 0)),
            pl.BlockSpec((1, 64), lambda i: (0, 0)),
            pl.BlockSpec((1, 64), lambda i: (0, 0)),
            pl.BlockSpec((1, 64), lambda i: (0, 0)),
            pl.BlockSpec((64, 128), lambda i: (0, 0)),
            pl.BlockSpec((1, 128), lambda i: (0, 0)),
        ],
        out_specs=[
            pl.BlockSpec((BE, 128), lambda i: (i, 0)),
            pl.BlockSpec((1, 128), lambda i: (0, 0)),
            pl.BlockSpec((1, 128), lambda i: (0, 0)),
        ],
        out_shape=[
            jax.ShapeDtypeStruct((N_EDGES, 128), f32),
            jax.ShapeDtypeStruct((1, 128), f32),
            jax.ShapeDtypeStruct((1, 128), f32),
        ],
        scratch_shapes=[pltpu.VMEM((1, 128), f32)] * 2,
    )(G1, G2, W1p, b1r, a1[None, :], c1[None, :], W2, b2[None, :])

    a2, c2 = _stats_to_affine(s2, q2, N_EDGES, g2, be2)

    aggraw = jnp.full((N_NODES, 128), NEG, f32)
    aggraw = aggraw.at[dst].max(h2)

    pos3p = jnp.concatenate([pos3, jnp.zeros((N_NODES, 5), f32)], axis=1)
    W3a = W3[:128]
    W3b = jnp.zeros((8, 128), f32).at[:3].set(W3[128:131])

    hp, s3, q3 = pl.pallas_call(
        _node_body,
        grid=(NNB,),
        in_specs=[
            pl.BlockSpec((BN_, 128), lambda i: (i, 0)),
            pl.BlockSpec((BN_, 8), lambda i: (i, 0)),
            pl.BlockSpec((128, 128), lambda i: (0, 0)),
            pl.BlockSpec((8, 128), lambda i: (0, 0)),
            pl.BlockSpec((1, 128), lambda i: (0, 0)),
            pl.BlockSpec((1, 128), lambda i: (0, 0)),
            pl.BlockSpec((1, 128), lambda i: (0, 0)),
        ],
        out_specs=[
            pl.BlockSpec((BN_, 128), lambda i: (i, 0)),
            pl.BlockSpec((1, 128), lambda i: (0, 0)),
            pl.BlockSpec((1, 128), lambda i: (0, 0)),
        ],
        out_shape=[
            jax.ShapeDtypeStruct((N_NODES, 128), f32),
            jax.ShapeDtypeStruct((1, 128), f32),
            jax.ShapeDtypeStruct((1, 128), f32),
        ],
        scratch_shapes=[pltpu.VMEM((1, 128), f32)] * 2,
    )(aggraw, pos3p, W3a, W3b, b3[None, :], a2[None, :], c2[None, :])

    a3, c3 = _stats_to_affine(s3, q3, N_NODES, g3, be3)

    W5p = jnp.zeros((64, 128), f32).at[:, 0].set(W5[:, 0])
    out = pl.pallas_call(
        _head_body,
        grid=(1,),
        in_specs=[
            pl.BlockSpec((N_NODES, 128), lambda i: (0, 0)),
            pl.BlockSpec((N_NODES, 1), lambda i: (0, 0)),
            pl.BlockSpec((1, 128), lambda i: (0, 0)),
            pl.BlockSpec((1, 128), lambda i: (0, 0)),
            pl.BlockSpec((128, 64), lambda i: (0, 0)),
            pl.BlockSpec((1, 64), lambda i: (0, 0)),
            pl.BlockSpec((1, 64), lambda i: (0, 0)),
            pl.BlockSpec((1, 64), lambda i: (0, 0)),
            pl.BlockSpec((64, 128), lambda i: (0, 0)),
        ],
        out_specs=pl.BlockSpec((N_GRAPHS, 128), lambda i: (0, 0)),
        out_shape=jax.ShapeDtypeStruct((N_GRAPHS, 128), f32),
    )(hp, batch[:, None], a3[None, :], c3[None, :], W4, b4[None, :],
      g4[None, :], be4[None, :], W5p)

    return out[:, 0:1] + b5[None, :]


# speedup vs baseline: 2.1906x; 1.6587x over previous
import functools
import jax
import jax.numpy as jnp
from jax import lax
from jax.experimental import pallas as pl
from jax.experimental.pallas import tpu as pltpu
from jax.experimental.pallas import tpu_sc as plsc

N_NODES = 10000
N_EDGES = 320000
N_GRAPHS = 16
EPS = 1e-5
NEG = -1e30

BE = 4000
NEB = N_EDGES // BE
BN_ = 1000
NNB = N_NODES // BN_


def _dotb(a, b):
    return jnp.dot(a.astype(jnp.bfloat16), b.astype(jnp.bfloat16),
                   preferred_element_type=jnp.float32)


def _pass1_body(g1_ref, g2_ref, w1_ref, b1_ref, s_ref, q_ref, acc_s, acc_q):
    step = pl.program_id(0)

    @pl.when(step == 0)
    def _():
        acc_s[...] = jnp.zeros_like(acc_s)
        acc_q[...] = jnp.zeros_like(acc_q)

    msg = g1_ref[...] - g2_ref[...]
    h = jnp.maximum(_dotb(msg, w1_ref[...]) + b1_ref[...], 0.0)
    acc_s[...] += jnp.sum(h, axis=0, keepdims=True)
    acc_q[...] += jnp.sum(h * h, axis=0, keepdims=True)

    @pl.when(step == NEB - 1)
    def _():
        s_ref[...] = acc_s[...]
        q_ref[...] = acc_q[...]


def _pass2_body(g1_ref, g2_ref, w1_ref, b1_ref, a1_ref, c1_ref, w2_ref,
                b2_ref, h2_ref, s_ref, q_ref, acc_s, acc_q):
    step = pl.program_id(0)

    @pl.when(step == 0)
    def _():
        acc_s[...] = jnp.zeros_like(acc_s)
        acc_q[...] = jnp.zeros_like(acc_q)

    msg = g1_ref[...] - g2_ref[...]
    h1 = jnp.maximum(_dotb(msg, w1_ref[...]) + b1_ref[...], 0.0)
    h1n = a1_ref[...] * h1 + c1_ref[...]
    h2 = jnp.maximum(_dotb(h1n, w2_ref[...]) + b2_ref[...], 0.0)
    h2_ref[...] = h2
    acc_s[...] += jnp.sum(h2, axis=0, keepdims=True)
    acc_q[...] += jnp.sum(h2 * h2, axis=0, keepdims=True)

    @pl.when(step == NEB - 1)
    def _():
        s_ref[...] = acc_s[...]
        q_ref[...] = acc_q[...]


def _node_body(agg_ref, p3_ref, w3a_ref, w3b_ref, b3_ref, a2_ref, c2_ref,
               hp_ref, s_ref, q_ref, acc_s, acc_q):
    step = pl.program_id(0)

    @pl.when(step == 0)
    def _():
        acc_s[...] = jnp.zeros_like(acc_s)
        acc_q[...] = jnp.zeros_like(acc_q)

    agg = agg_ref[...]
    aggn = jnp.where(agg > -1e29, a2_ref[...] * agg + c2_ref[...], 0.0)
    pre = _dotb(aggn, w3a_ref[...]) + _dotb(p3_ref[...], w3b_ref[...])
    hp = jnp.maximum(pre + b3_ref[...], 0.0)
    hp_ref[...] = hp
    acc_s[...] += jnp.sum(hp, axis=0, keepdims=True)
    acc_q[...] += jnp.sum(hp * hp, axis=0, keepdims=True)

    @pl.when(step == NNB - 1)
    def _():
        s_ref[...] = acc_s[...]
        q_ref[...] = acc_q[...]


def _head_body(hp_ref, b_ref, a3_ref, c3_ref, w4_ref, b4_ref, g4_ref,
               be4_ref, w5_ref, out_ref):
    hpn = a3_ref[...] * hp_ref[...] + c3_ref[...]
    bid = b_ref[...]
    rows = []
    for g in range(N_GRAPHS):
        rows.append(jnp.where(bid == g, hpn, NEG).max(axis=0, keepdims=True))
    gp = jnp.concatenate(rows, axis=0)
    gp = jnp.where(gp > -1e29, gp, 0.0)
    h4 = jnp.maximum(_dotb(gp, w4_ref[...]) + b4_ref[...], 0.0)
    m4 = jnp.mean(h4, axis=0, keepdims=True)
    v4 = jnp.mean((h4 - m4) * (h4 - m4), axis=0, keepdims=True)
    h4n = g4_ref[...] * (h4 - m4) / jnp.sqrt(v4 + EPS) + be4_ref[...]
    out_ref[...] = _dotb(h4n, w5_ref[...])


NW = 32
EPW = N_EDGES // NW
GCH = 80
GFIRE = 5
GOUT = GCH * GFIRE
GITER = EPW // GOUT


def _sc_gather_body(src_hbm, dst_hbm, t1_hbm, t2_hbm, g1_hbm, g2_hbm,
                    idx1_v, idx2_v, rows1_v, rows2_v, sem):
    wid = lax.axis_index("s") * 2 + lax.axis_index("c")
    wbase = wid * EPW

    def outer(i, _):
        base = wbase + i * GOUT
        pltpu.sync_copy(src_hbm.at[pl.ds(base, GOUT)], idx1_v)
        pltpu.sync_copy(dst_hbm.at[pl.ds(base, GOUT)], idx2_v)
        copies = []
        for j in range(GFIRE):
            copies.append(pltpu.async_copy(
                t1_hbm.at[idx1_v.at[pl.ds(j * GCH, GCH)]],
                rows1_v.at[pl.ds(j * GCH, GCH)], sem))
            copies.append(pltpu.async_copy(
                t2_hbm.at[idx2_v.at[pl.ds(j * GCH, GCH)]],
                rows2_v.at[pl.ds(j * GCH, GCH)], sem))
        for c in copies:
            c.wait()
        pltpu.sync_copy(rows1_v, g1_hbm.at[pl.ds(base, GOUT)])
        pltpu.sync_copy(rows2_v, g2_hbm.at[pl.ds(base, GOUT)])
        return 0

    lax.fori_loop(0, GITER, outer, 0)


def _sc_gather(src, dst, T1, T2):
    f32 = jnp.float32
    k = pl.kernel(
        _sc_gather_body,
        out_type=[jax.ShapeDtypeStruct((N_EDGES, 16), f32)] * 2,
        mesh=plsc.VectorSubcoreMesh(core_axis_name="c", subcore_axis_name="s"),
        compiler_params=pltpu.CompilerParams(use_tc_tiling_on_sc=False),
        scratch_types=[
            pltpu.VMEM((GOUT,), jnp.int32),
            pltpu.VMEM((GOUT,), jnp.int32),
            pltpu.VMEM((GOUT, 16), f32),
            pltpu.VMEM((GOUT, 16), f32),
            pltpu.SemaphoreType.DMA,
        ],
    )
    return k(src, dst, T1, T2)


def _stats_to_affine(s, q, n, g, be):
    mean = s.sum(0) / n
    var = q.sum(0) / n - mean * mean
    a = g / jnp.sqrt(var + EPS)
    c = be - a * mean
    return a, c


def kernel(x, pos, batch, edge_index, W1, b1, g1, be1, W2, b2, g2, be2,
           W3, b3, g3, be3, W4, b4, g4, be4, W5, b5):
    f32 = jnp.float32
    src = edge_index[0]
    dst = edge_index[1]

    pos3 = jnp.concatenate(
        [jnp.cos(pos[:, 1:2]), jnp.sin(pos[:, 1:2]), pos[:, 0:1]], axis=-1)
    zeros3 = jnp.zeros((N_NODES, 3), f32)
    pad10 = jnp.zeros((N_NODES, 10), f32)
    T1 = jnp.concatenate([x, pos3, pad10], axis=1)
    T2 = jnp.concatenate([zeros3, pos3, pad10], axis=1)
    G1, G2 = _sc_gather(src, dst, T1, T2)

    W1p = jnp.zeros((16, 64), f32).at[:6].set(W1)
    b1r = b1[None, :]

    s1, q1 = pl.pallas_call(
        _pass1_body,
        grid=(NEB,),
        in_specs=[
            pl.BlockSpec((BE, 16), lambda i: (i, 0)),
            pl.BlockSpec((BE, 16), lambda i: (i, 0)),
            pl.BlockSpec((16, 64), lambda i: (0, 0)),
            pl.BlockSpec((1, 64), lambda i: (0, 0)),
        ],
        out_specs=[
            pl.BlockSpec((1, 64), lambda i: (0, 0)),
            pl.BlockSpec((1, 64), lambda i: (0, 0)),
        ],
        out_shape=[jax.ShapeDtypeStruct((1, 64), f32)] * 2,
        scratch_shapes=[pltpu.VMEM((1, 64), f32)] * 2,
    )(G1, G2, W1p, b1r)

    a1, c1 = _stats_to_affine(s1, q1, N_EDGES, g1, be1)

    h2, s2, q2 = pl.pallas_call(
        _pass2_body,
        grid=(NEB,),
        in_specs=[
            pl.BlockSpec((BE, 16), lambda i: (i, 0)),
            pl.BlockSpec((BE, 16), lambda i: (i, 0)),
            pl.BlockSpec((16, 64), lambda i: (0, 0)),
            pl.BlockSpec((1, 64), lambda i: (0, 0)),
            pl.BlockSpec((1, 64), lambda i: (0, 0)),
            pl.BlockSpec((1, 64), lambda i: (0, 0)),
            pl.BlockSpec((64, 128), lambda i: (0, 0)),
            pl.BlockSpec((1, 128), lambda i: (0, 0)),
        ],
        out_specs=[
            pl.BlockSpec((BE, 128), lambda i: (i, 0)),
            pl.BlockSpec((1, 128), lambda i: (0, 0)),
            pl.BlockSpec((1, 128), lambda i: (0, 0)),
        ],
        out_shape=[
            jax.ShapeDtypeStruct((N_EDGES, 128), f32),
            jax.ShapeDtypeStruct((1, 128), f32),
            jax.ShapeDtypeStruct((1, 128), f32),
        ],
        scratch_shapes=[pltpu.VMEM((1, 128), f32)] * 2,
    )(G1, G2, W1p, b1r, a1[None, :], c1[None, :], W2, b2[None, :])

    a2, c2 = _stats_to_affine(s2, q2, N_EDGES, g2, be2)

    aggraw = jnp.full((N_NODES, 128), NEG, f32)
    aggraw = aggraw.at[dst].max(h2)

    pos3p = jnp.concatenate([pos3, jnp.zeros((N_NODES, 5), f32)], axis=1)
    W3a = W3[:128]
    W3b = jnp.zeros((8, 128), f32).at[:3].set(W3[128:131])

    hp, s3, q3 = pl.pallas_call(
        _node_body,
        grid=(NNB,),
        in_specs=[
            pl.BlockSpec((BN_, 128), lambda i: (i, 0)),
            pl.BlockSpec((BN_, 8), lambda i: (i, 0)),
            pl.BlockSpec((128, 128), lambda i: (0, 0)),
            pl.BlockSpec((8, 128), lambda i: (0, 0)),
            pl.BlockSpec((1, 128), lambda i: (0, 0)),
            pl.BlockSpec((1, 128), lambda i: (0, 0)),
            pl.BlockSpec((1, 128), lambda i: (0, 0)),
        ],
        out_specs=[
            pl.BlockSpec((BN_, 128), lambda i: (i, 0)),
            pl.BlockSpec((1, 128), lambda i: (0, 0)),
            pl.BlockSpec((1, 128), lambda i: (0, 0)),
        ],
        out_shape=[
            jax.ShapeDtypeStruct((N_NODES, 128), f32),
            jax.ShapeDtypeStruct((1, 128), f32),
            jax.ShapeDtypeStruct((1, 128), f32),
        ],
        scratch_shapes=[pltpu.VMEM((1, 128), f32)] * 2,
    )(aggraw, pos3p, W3a, W3b, b3[None, :], a2[None, :], c2[None, :])

    a3, c3 = _stats_to_affine(s3, q3, N_NODES, g3, be3)

    W5p = jnp.zeros((64, 128), f32).at[:, 0].set(W5[:, 0])
    out = pl.pallas_call(
        _head_body,
        grid=(1,),
        in_specs=[
            pl.BlockSpec((N_NODES, 128), lambda i: (0, 0)),
            pl.BlockSpec((N_NODES, 1), lambda i: (0, 0)),
            pl.BlockSpec((1, 128), lambda i: (0, 0)),
            pl.BlockSpec((1, 128), lambda i: (0, 0)),
            pl.BlockSpec((128, 64), lambda i: (0, 0)),
            pl.BlockSpec((1, 64), lambda i: (0, 0)),
            pl.BlockSpec((1, 64), lambda i: (0, 0)),
            pl.BlockSpec((1, 64), lambda i: (0, 0)),
            pl.BlockSpec((64, 128), lambda i: (0, 0)),
        ],
        out_specs=pl.BlockSpec((N_GRAPHS, 128), lambda i: (0, 0)),
        out_shape=jax.ShapeDtypeStruct((N_GRAPHS, 128), f32),
    )(hp, batch[:, None], a3[None, :], c3[None, :], W4, b4[None, :],
      g4[None, :], be4[None, :], W5p)

    return out[:, 0:1] + b5[None, :]


# speedup vs baseline: 2.1926x; 1.0009x over previous
import functools
import jax
import jax.numpy as jnp
from jax import lax
from jax.experimental import pallas as pl
from jax.experimental.pallas import tpu as pltpu
from jax.experimental.pallas import tpu_sc as plsc

N_NODES = 10000
N_EDGES = 320000
N_GRAPHS = 16
EPS = 1e-5
NEG = -1e30

BE = 4000
NEB = N_EDGES // BE
BN_ = 1000
NNB = N_NODES // BN_


def _dotb(a, b):
    return jnp.dot(a.astype(jnp.bfloat16), b.astype(jnp.bfloat16),
                   preferred_element_type=jnp.float32)


def _pass1_body(g1_ref, g2_ref, w1_ref, b1_ref, s_ref, q_ref, acc_s, acc_q):
    step = pl.program_id(0)

    @pl.when(step == 0)
    def _():
        acc_s[...] = jnp.zeros_like(acc_s)
        acc_q[...] = jnp.zeros_like(acc_q)

    msg = g1_ref[...] - g2_ref[...]
    h = jnp.maximum(_dotb(msg, w1_ref[...]) + b1_ref[...], 0.0)
    acc_s[...] += jnp.sum(h, axis=0, keepdims=True)
    acc_q[...] += jnp.sum(h * h, axis=0, keepdims=True)

    @pl.when(step == NEB - 1)
    def _():
        s_ref[...] = acc_s[...]
        q_ref[...] = acc_q[...]


def _pass2_body(g1_ref, g2_ref, w1_ref, b1_ref, a1_ref, c1_ref, w2_ref,
                b2_ref, h2_ref, s_ref, q_ref, acc_s, acc_q):
    step = pl.program_id(0)

    @pl.when(step == 0)
    def _():
        acc_s[...] = jnp.zeros_like(acc_s)
        acc_q[...] = jnp.zeros_like(acc_q)

    msg = g1_ref[...] - g2_ref[...]
    h1 = jnp.maximum(_dotb(msg, w1_ref[...]) + b1_ref[...], 0.0)
    h1n = a1_ref[...] * h1 + c1_ref[...]
    h2 = jnp.maximum(_dotb(h1n, w2_ref[...]) + b2_ref[...], 0.0)
    h2_ref[...] = h2
    acc_s[...] += jnp.sum(h2, axis=0, keepdims=True)
    acc_q[...] += jnp.sum(h2 * h2, axis=0, keepdims=True)

    @pl.when(step == NEB - 1)
    def _():
        s_ref[...] = acc_s[...]
        q_ref[...] = acc_q[...]


def _node_body(agg_ref, p3_ref, w3a_ref, w3b_ref, b3_ref, a2_ref, c2_ref,
               hp_ref, s_ref, q_ref, acc_s, acc_q):
    step = pl.program_id(0)

    @pl.when(step == 0)
    def _():
        acc_s[...] = jnp.zeros_like(acc_s)
        acc_q[...] = jnp.zeros_like(acc_q)

    agg = agg_ref[...]
    aggn = jnp.where(agg > -1e29, a2_ref[...] * agg + c2_ref[...], 0.0)
    pre = _dotb(aggn, w3a_ref[...]) + _dotb(p3_ref[...], w3b_ref[...])
    hp = jnp.maximum(pre + b3_ref[...], 0.0)
    hp_ref[...] = hp
    acc_s[...] += jnp.sum(hp, axis=0, keepdims=True)
    acc_q[...] += jnp.sum(hp * hp, axis=0, keepdims=True)

    @pl.when(step == NNB - 1)
    def _():
        s_ref[...] = acc_s[...]
        q_ref[...] = acc_q[...]


def _head_body(hp_ref, b_ref, a3_ref, c3_ref, w4_ref, b4_ref, g4_ref,
               be4_ref, w5_ref, out_ref):
    hpn = a3_ref[...] * hp_ref[...] + c3_ref[...]
    bid = b_ref[...]
    rows = []
    for g in range(N_GRAPHS):
        rows.append(jnp.where(bid == g, hpn, NEG).max(axis=0, keepdims=True))
    gp = jnp.concatenate(rows, axis=0)
    gp = jnp.where(gp > -1e29, gp, 0.0)
    h4 = jnp.maximum(_dotb(gp, w4_ref[...]) + b4_ref[...], 0.0)
    m4 = jnp.mean(h4, axis=0, keepdims=True)
    v4 = jnp.mean((h4 - m4) * (h4 - m4), axis=0, keepdims=True)
    h4n = g4_ref[...] * (h4 - m4) / jnp.sqrt(v4 + EPS) + be4_ref[...]
    out_ref[...] = _dotb(h4n, w5_ref[...])


NW = 32
EPW = N_EDGES // NW
GCH = 80
GFIRE = 5
GOUT = GCH * GFIRE
GITER = EPW // GOUT


def _sc_gather_body(src_hbm, dst_hbm, t1_hbm, t2_hbm, g1_hbm, g2_hbm,
                    idx1_v, idx2_v, rows1_v, rows2_v, sem):
    wid = lax.axis_index("s") * 2 + lax.axis_index("c")
    wbase = wid * EPW

    def outer(i, _):
        base = wbase + i * GOUT
        pltpu.sync_copy(src_hbm.at[pl.ds(base, GOUT)], idx1_v)
        pltpu.sync_copy(dst_hbm.at[pl.ds(base, GOUT)], idx2_v)
        copies = []
        for j in range(GFIRE):
            copies.append(pltpu.async_copy(
                t1_hbm.at[idx1_v.at[pl.ds(j * GCH, GCH)]],
                rows1_v.at[pl.ds(j * GCH, GCH)], sem))
            copies.append(pltpu.async_copy(
                t2_hbm.at[idx2_v.at[pl.ds(j * GCH, GCH)]],
                rows2_v.at[pl.ds(j * GCH, GCH)], sem))
        for c in copies:
            c.wait()
        pltpu.sync_copy(rows1_v, g1_hbm.at[pl.ds(base, GOUT)])
        pltpu.sync_copy(rows2_v, g2_hbm.at[pl.ds(base, GOUT)])
        return 0

    lax.fori_loop(0, GITER, outer, 0)


def _sc_gather(src, dst, T1, T2):
    f32 = jnp.float32
    k = pl.kernel(
        _sc_gather_body,
        out_type=[jax.ShapeDtypeStruct((N_EDGES, 16), f32)] * 2,
        mesh=plsc.VectorSubcoreMesh(core_axis_name="c", subcore_axis_name="s"),
        compiler_params=pltpu.CompilerParams(use_tc_tiling_on_sc=False),
        scratch_types=[
            pltpu.VMEM((GOUT,), jnp.int32),
            pltpu.VMEM((GOUT,), jnp.int32),
            pltpu.VMEM((GOUT, 16), f32),
            pltpu.VMEM((GOUT, 16), f32),
            pltpu.SemaphoreType.DMA,
        ],
    )
    return k(src, dst, T1, T2)


NPW = 320
N_PAD = NW * NPW
ET = 8000
NTILES = N_EDGES // ET
LCAP = ET + 128


def _sc_scatter_body(dst_hbm, h2_hbm, out_hbm,
                     acc_v, dstb_v, leid_v, loff_v, rows_v, sem):
    wid = lax.axis_index("s") * 2 + lax.axis_index("c")
    lo = wid * NPW
    neg = jnp.full((16,), NEG, jnp.float32)
    lanes = lax.iota(jnp.int32, 16)

    def init(i, _):
        acc_v[pl.ds(i * 16, 16)] = neg
        return 0
    lax.fori_loop(0, NPW * 128 // 16, init, 0)

    def tile(t, _):
        pltpu.sync_copy(dst_hbm.at[pl.ds(t * ET, ET)], dstb_v)

        def scan(c, cnt_v):
            d = dstb_v[pl.ds(c * 16, 16)]
            dl = d - lo
            m = (dl >= 0) & (dl < NPW)
            inc = plsc.cumsum(m.astype(jnp.int32))
            pos = cnt_v + inc - 1
            eid = t * ET + c * 16 + lanes
            plsc.store_scatter(leid_v, [pos], eid, mask=m)
            plsc.store_scatter(loff_v, [pos], dl * 128, mask=m)
            return cnt_v + plsc.all_reduce_population_count(m)

        cnt_v = lax.fori_loop(0, ET // 16, scan,
                              jnp.zeros((16,), jnp.int32))
        cnt = lax.reduce_max(cnt_v, (0,))

        @pl.when(cnt > 0)
        def _():
            eid_l = jnp.full((16,), leid_v[pl.ds(0, 16)][0], jnp.int32)
            off_l = jnp.full((16,), loff_v[pl.ds(0, 16)][0], jnp.int32)
            for j in range(8):
                ppos = cnt_v + j * 16 + lanes
                plsc.store_scatter(leid_v, [ppos], eid_l)
                plsc.store_scatter(loff_v, [ppos], off_l)

            def group(g, _):
                pltpu.async_copy(
                    h2_hbm.at[leid_v.at[pl.ds(g * 128, 128)]],
                    rows_v, sem).wait()

                def rowgrp(rr, _):
                    offv = loff_v[pl.ds(g * 128 + rr * 16, 16)]
                    for r16 in range(16):
                        off = offv[r16]
                        row = rr * 16 + r16
                        for k in range(8):
                            sl = pl.ds(off + k * 16, 16)
                            acc_v[sl] = jnp.maximum(
                                acc_v[sl], rows_v[row, pl.ds(k * 16, 16)])
                    return 0
                lax.fori_loop(0, 8, rowgrp, 0)
                return 0

            lax.fori_loop(0, (cnt + 127) // 128, group, 0)
        return 0

    lax.fori_loop(0, NTILES, tile, 0)
    pltpu.sync_copy(acc_v, out_hbm.at[pl.ds(lo * 128, NPW * 128)])


def _sc_scatter_max(dst, h2):
    k = pl.kernel(
        _sc_scatter_body,
        out_type=jax.ShapeDtypeStruct((N_PAD * 128,), jnp.float32),
        mesh=plsc.VectorSubcoreMesh(core_axis_name="c", subcore_axis_name="s"),
        compiler_params=pltpu.CompilerParams(use_tc_tiling_on_sc=False),
        scratch_types=[
            pltpu.VMEM((NPW * 128,), jnp.float32),
            pltpu.VMEM((ET,), jnp.int32),
            pltpu.VMEM((LCAP,), jnp.int32),
            pltpu.VMEM((LCAP,), jnp.int32),
            pltpu.VMEM((128, 128), jnp.float32),
            pltpu.SemaphoreType.DMA,
        ],
    )
    return k(dst, h2).reshape(N_PAD, 128)[:N_NODES]


def _stats_to_affine(s, q, n, g, be):
    mean = s.sum(0) / n
    var = q.sum(0) / n - mean * mean
    a = g / jnp.sqrt(var + EPS)
    c = be - a * mean
    return a, c


def kernel(x, pos, batch, edge_index, W1, b1, g1, be1, W2, b2, g2, be2,
           W3, b3, g3, be3, W4, b4, g4, be4, W5, b5):
    f32 = jnp.float32
    src = edge_index[0]
    dst = edge_index[1]

    pos3 = jnp.concatenate(
        [jnp.cos(pos[:, 1:2]), jnp.sin(pos[:, 1:2]), pos[:, 0:1]], axis=-1)
    zeros3 = jnp.zeros((N_NODES, 3), f32)
    pad10 = jnp.zeros((N_NODES, 10), f32)
    T1 = jnp.concatenate([x, pos3, pad10], axis=1)
    T2 = jnp.concatenate([zeros3, pos3, pad10], axis=1)
    G1, G2 = _sc_gather(src, dst, T1, T2)

    W1p = jnp.zeros((16, 64), f32).at[:6].set(W1)
    b1r = b1[None, :]

    s1, q1 = pl.pallas_call(
        _pass1_body,
        grid=(NEB,),
        in_specs=[
            pl.BlockSpec((BE, 16), lambda i: (i, 0)),
            pl.BlockSpec((BE, 16), lambda i: (i, 0)),
            pl.BlockSpec((16, 64), lambda i: (0, 0)),
            pl.BlockSpec((1, 64), lambda i: (0, 0)),
        ],
        out_specs=[
            pl.BlockSpec((1, 64), lambda i: (0, 0)),
            pl.BlockSpec((1, 64), lambda i: (0, 0)),
        ],
        out_shape=[jax.ShapeDtypeStruct((1, 64), f32)] * 2,
        scratch_shapes=[pltpu.VMEM((1, 64), f32)] * 2,
    )(G1, G2, W1p, b1r)

    a1, c1 = _stats_to_affine(s1, q1, N_EDGES, g1, be1)

    h2, s2, q2 = pl.pallas_call(
        _pass2_body,
        grid=(NEB,),
        in_specs=[
            pl.BlockSpec((BE, 16), lambda i: (i, 0)),
            pl.BlockSpec((BE, 16), lambda i: (i, 0)),
            pl.BlockSpec((16, 64), lambda i: (0, 0)),
            pl.BlockSpec((1, 64), lambda i: (0, 0)),
            pl.BlockSpec((1, 64), lambda i: (0, 0)),
            pl.BlockSpec((1, 64), lambda i: (0, 0)),
            pl.BlockSpec((64, 128), lambda i: (0, 0)),
            pl.BlockSpec((1, 128), lambda i: (0, 0)),
        ],
        out_specs=[
            pl.BlockSpec((BE, 128), lambda i: (i, 0)),
            pl.BlockSpec((1, 128), lambda i: (0, 0)),
            pl.BlockSpec((1, 128), lambda i: (0, 0)),
        ],
        out_shape=[
            jax.ShapeDtypeStruct((N_EDGES, 128), f32),
            jax.ShapeDtypeStruct((1, 128), f32),
            jax.ShapeDtypeStruct((1, 128), f32),
        ],
        scratch_shapes=[pltpu.VMEM((1, 128), f32)] * 2,
    )(G1, G2, W1p, b1r, a1[None, :], c1[None, :], W2, b2[None, :])

    a2, c2 = _stats_to_affine(s2, q2, N_EDGES, g2, be2)

    aggraw = jnp.full((N_NODES, 128), NEG, f32).at[dst].max(h2)

    pos3p = jnp.concatenate([pos3, jnp.zeros((N_NODES, 5), f32)], axis=1)
    W3a = W3[:128]
    W3b = jnp.zeros((8, 128), f32).at[:3].set(W3[128:131])

    hp, s3, q3 = pl.pallas_call(
        _node_body,
        grid=(NNB,),
        in_specs=[
            pl.BlockSpec((BN_, 128), lambda i: (i, 0)),
            pl.BlockSpec((BN_, 8), lambda i: (i, 0)),
            pl.BlockSpec((128, 128), lambda i: (0, 0)),
            pl.BlockSpec((8, 128), lambda i: (0, 0)),
            pl.BlockSpec((1, 128), lambda i: (0, 0)),
            pl.BlockSpec((1, 128), lambda i: (0, 0)),
            pl.BlockSpec((1, 128), lambda i: (0, 0)),
        ],
        out_specs=[
            pl.BlockSpec((BN_, 128), lambda i: (i, 0)),
            pl.BlockSpec((1, 128), lambda i: (0, 0)),
            pl.BlockSpec((1, 128), lambda i: (0, 0)),
        ],
        out_shape=[
            jax.ShapeDtypeStruct((N_NODES, 128), f32),
            jax.ShapeDtypeStruct((1, 128), f32),
            jax.ShapeDtypeStruct((1, 128), f32),
        ],
        scratch_shapes=[pltpu.VMEM((1, 128), f32)] * 2,
    )(aggraw, pos3p, W3a, W3b, b3[None, :], a2[None, :], c2[None, :])

    a3, c3 = _stats_to_affine(s3, q3, N_NODES, g3, be3)

    W5p = jnp.zeros((64, 128), f32).at[:, 0].set(W5[:, 0])
    out = pl.pallas_call(
        _head_body,
        grid=(1,),
        in_specs=[
            pl.BlockSpec((N_NODES, 128), lambda i: (0, 0)),
            pl.BlockSpec((N_NODES, 1), lambda i: (0, 0)),
            pl.BlockSpec((1, 128), lambda i: (0, 0)),
            pl.BlockSpec((1, 128), lambda i: (0, 0)),
            pl.BlockSpec((128, 64), lambda i: (0, 0)),
            pl.BlockSpec((1, 64), lambda i: (0, 0)),
            pl.BlockSpec((1, 64), lambda i: (0, 0)),
            pl.BlockSpec((1, 64), lambda i: (0, 0)),
            pl.BlockSpec((64, 128), lambda i: (0, 0)),
        ],
        out_specs=pl.BlockSpec((N_GRAPHS, 128), lambda i: (0, 0)),
        out_shape=jax.ShapeDtypeStruct((N_GRAPHS, 128), f32),
    )(hp, batch[:, None], a3[None, :], c3[None, :], W4, b4[None, :],
      g4[None, :], be4[None, :], W5p)

    return out[:, 0:1] + b5[None, :]
